# baseline (speedup 1.0000x reference)
import functools
import jax
import jax.numpy as jnp
from jax import lax
from jax.experimental import pallas as pl
from jax.experimental.pallas import tpu as pltpu
from jax.experimental.pallas import tpu_sc as plsc

_CH = 256
_NW = 32


def _make_sc(N, C, Dd, Dt, Vd, Vt):
    per_w = N // _NW
    n_chunks = per_w // _CH
    W = C + Dd + Dt
    mesh = plsc.VectorSubcoreMesh(core_axis_name="c", subcore_axis_name="s")

    @functools.partial(
        pl.kernel,
        mesh=mesh,
        compiler_params=pltpu.CompilerParams(needs_layout_passes=False),
        out_type=jax.ShapeDtypeStruct((N, W), jnp.float32),
        scratch_types=[
            pltpu.VMEM((_CH,), jnp.int32),
            pltpu.VMEM((_CH, W), jnp.float32),
            pltpu.VMEM((Vd, Dd), jnp.float32),
            pltpu.VMEM((Vt, Dt), jnp.float32),
        ],
    )
    def k(inp_hbm, cid_hbm, wd_hbm, wt_hbm, out_hbm, idx_v, out_v, wd_v, wt_v):
        wid = lax.axis_index("s") * 2 + lax.axis_index("c")
        w_base = pl.multiple_of(wid * per_w, _CH)
        pltpu.sync_copy(wd_hbm, wd_v)
        pltpu.sync_copy(wt_hbm, wt_v)

        def chunk(g, carry):
            base = pl.multiple_of(w_base + g * _CH, _CH)
            pltpu.sync_copy(cid_hbm.at[pl.ds(base, _CH)], idx_v)
            pltpu.sync_copy(inp_hbm.at[pl.ds(base, _CH)],
                            out_v.at[:, pl.ds(0, C)])
            def embed_grp(grp, c2):
                c = idx_v[pl.ds(grp * 16, 16)]
                d = c >> 9
                t = c & 511
                rows = lax.broadcasted_iota(jnp.int32, (16,), 0) + grp * 16
                for col in range(Dd):
                    v = plsc.load_gather(
                        wd_v, [d, jnp.full((16,), col, jnp.int32)])
                    plsc.store_scatter(
                        out_v, [rows, jnp.full((16,), C + col, jnp.int32)], v)
                for col in range(Dt):
                    v = plsc.load_gather(
                        wt_v, [t, jnp.full((16,), col, jnp.int32)])
                    plsc.store_scatter(
                        out_v,
                        [rows, jnp.full((16,), C + Dd + col, jnp.int32)], v)
                return c2

            lax.fori_loop(0, _CH // 16, embed_grp, 0)
            pltpu.sync_copy(out_v, out_hbm.at[pl.ds(base, _CH)])
            return carry

        lax.fori_loop(0, n_chunks, chunk, 0)

    return k


def kernel(inp, daytime, W_day, W_time):
    B, T, C = inp.shape
    N = B * T
    Dd, Dt = W_day.shape[1], W_time.shape[1]
    dt32 = daytime.astype(jnp.int32)
    cid = ((dt32[:, :, 0] << 9) | dt32[:, :, 1]).reshape(N)
    inp2 = inp.reshape(N, C)
    out = _make_sc(N, C, Dd, Dt, W_day.shape[0], W_time.shape[0])(
        inp2, cid, W_day, W_time)
    return out.reshape(B, T, C + Dd + Dt)


# speedup vs baseline: 1.3441x; 1.3441x over previous
import functools
import jax
import jax.numpy as jnp
from jax import lax
from jax.experimental import pallas as pl
from jax.experimental.pallas import tpu as pltpu
from jax.experimental.pallas import tpu_sc as plsc

_CB = 4
_NW = 32


def _make_sc(B, T, C, Dd, Dt, Vd, Vt):
    rows_w = B // _NW
    n_chunks = rows_w // _CB
    tok_w = rows_w * T
    tok_c = _CB * T
    W = C + Dd + Dt
    mesh = plsc.VectorSubcoreMesh(core_axis_name="c", subcore_axis_name="s")

    @functools.partial(
        pl.kernel,
        mesh=mesh,
        compiler_params=pltpu.CompilerParams(needs_layout_passes=False),
        out_type=jax.ShapeDtypeStruct((B, T, W), jnp.float32),
        scratch_types=[
            pltpu.VMEM((tok_w,), jnp.int32),
            pltpu.VMEM((_CB, T, W), jnp.float32),
            pltpu.VMEM((Vd, Dd), jnp.float32),
            pltpu.VMEM((Vt, Dt), jnp.float32),
        ],
    )
    def k(inp_hbm, cid_hbm, wd_hbm, wt_hbm, out_hbm, idx_v, out_v, wd_v, wt_v):
        wid = lax.axis_index("s") * 2 + lax.axis_index("c")
        b_base = pl.multiple_of(wid * rows_w, _CB)
        pltpu.sync_copy(wd_hbm, wd_v)
        pltpu.sync_copy(wt_hbm, wt_v)
        pltpu.sync_copy(cid_hbm.at[pl.ds(pl.multiple_of(wid * tok_w, 128),
                                         tok_w)], idx_v)

        def chunk(g, carry):
            b0 = pl.multiple_of(b_base + g * _CB, _CB)
            pltpu.sync_copy(inp_hbm.at[pl.ds(b0, _CB)],
                            out_v.at[:, :, pl.ds(0, C)])

            def embed_grp(grp, c2):
                l0 = grp * 16
                c = idx_v[pl.ds(g * tok_c + l0, 16)]
                d = c >> 9
                t = c & 511
                l = lax.broadcasted_iota(jnp.int32, (16,), 0) + l0
                b_l = lax.div(l, T)
                t_l = lax.rem(l, T)
                for col in range(Dd):
                    v = plsc.load_gather(
                        wd_v, [d, jnp.full((16,), col, jnp.int32)])
                    plsc.store_scatter(
                        out_v,
                        [b_l, t_l, jnp.full((16,), C + col, jnp.int32)], v)
                for col in range(Dt):
                    v = plsc.load_gather(
                        wt_v, [t, jnp.full((16,), col, jnp.int32)])
                    plsc.store_scatter(
                        out_v,
                        [b_l, t_l, jnp.full((16,), C + Dd + col, jnp.int32)],
                        v)
                return c2

            lax.fori_loop(0, tok_c // 16, embed_grp, 0)
            pltpu.sync_copy(out_v, out_hbm.at[pl.ds(b0, _CB)])
            return carry

        lax.fori_loop(0, n_chunks, chunk, 0)

    return k


def kernel(inp, daytime, W_day, W_time):
    B, T, C = inp.shape
    Dd, Dt = W_day.shape[1], W_time.shape[1]
    dt32 = daytime.astype(jnp.int32)
    cid = ((dt32[:, :, 0] << 9) | dt32[:, :, 1]).reshape(B * T)
    return _make_sc(B, T, C, Dd, Dt, W_day.shape[0], W_time.shape[0])(
        inp, cid, W_day, W_time)
